# baseline (speedup 1.0000x reference)
import functools

import jax
import jax.numpy as jnp
from jax import lax
from jax.experimental import pallas as pl
from jax.experimental.pallas import tpu as pltpu
from jax.experimental.pallas import tpu_sc as plsc

MAX_LEN = 8192
D = 1024
NC, NS, L = 2, 16, 16
NW = NC * NS
B_TOTAL = 4 * 8192
B_PER_W = B_TOTAL // NW
CHUNK = 32
N_CHUNKS = B_PER_W // CHUNK
NBUF = 3


def _gather_kernel(table_hbm, idx_hbm, out_hbm, idx_v, rows_v, *sems):
    wid = lax.axis_index("s") * NC + lax.axis_index("c")
    base = wid * B_PER_W

    pltpu.sync_copy(idx_hbm.at[wid], idx_v)
    for g in range(N_CHUNKS):
        for j in range(CHUNK // L):
            v = idx_v[g, pl.ds(j * L, L)]
            v = lax.max(lax.min(v, MAX_LEN - 1), 0)
            idx_v[g, pl.ds(j * L, L)] = v

    gsems = sems[:NBUF]
    osems = sems[NBUF:]
    gathers = [None] * NBUF
    writes = [None] * NBUF
    for g in range(N_CHUNKS):
        b = g % NBUF
        if writes[b] is not None:
            writes[b].wait()
            writes[b] = None
        gathers[b] = pltpu.async_copy(
            table_hbm.at[idx_v.at[g]], rows_v.at[b], gsems[b])
        if g >= 1:
            pb = (g - 1) % NBUF
            gathers[pb].wait()
            writes[pb] = pltpu.async_copy(
                rows_v.at[pb], out_hbm.at[pl.ds(base + (g - 1) * CHUNK, CHUNK)],
                osems[pb])
    lb = (N_CHUNKS - 1) % NBUF
    gathers[lb].wait()
    writes[lb] = pltpu.async_copy(
        rows_v.at[lb],
        out_hbm.at[pl.ds(base + (N_CHUNKS - 1) * CHUNK, CHUNK)], osems[lb])
    for b in range(NBUF):
        if writes[b] is not None:
            writes[b].wait()


@jax.jit
def kernel(position_ids, table):
    idx = position_ids.reshape(NW, N_CHUNKS, CHUNK).astype(jnp.int32)
    run = pl.kernel(
        _gather_kernel,
        out_type=jax.ShapeDtypeStruct((B_TOTAL, D), jnp.float32),
        mesh=plsc.VectorSubcoreMesh(core_axis_name="c", subcore_axis_name="s"),
        scratch_types=[
            pltpu.VMEM((N_CHUNKS, CHUNK), jnp.int32),
            pltpu.VMEM((NBUF, CHUNK, D), jnp.float32),
        ] + [pltpu.SemaphoreType.DMA] * (2 * NBUF),
    )
    out = run(table, idx)
    return out.reshape(position_ids.shape[0], position_ids.shape[1], D)


# speedup vs baseline: 1.0100x; 1.0100x over previous
import functools

import jax
import jax.numpy as jnp
from jax import lax
from jax.experimental import pallas as pl
from jax.experimental.pallas import tpu as pltpu
from jax.experimental.pallas import tpu_sc as plsc

MAX_LEN = 8192
D = 1024
NC, NS, L = 2, 16, 16
NW = NC * NS
B_TOTAL = 4 * 8192
B_PER_W = B_TOTAL // NW
CHUNK = 16
N_CHUNKS = B_PER_W // CHUNK
NBUF = 7


def _gather_kernel(table_hbm, idx_hbm, out_hbm, idx_v, rows_v, *sems):
    wid = lax.axis_index("s") * NC + lax.axis_index("c")
    base = wid * B_PER_W

    pltpu.sync_copy(idx_hbm.at[wid], idx_v)
    for g in range(N_CHUNKS):
        for j in range(CHUNK // L):
            v = idx_v[g, pl.ds(j * L, L)]
            v = lax.max(lax.min(v, MAX_LEN - 1), 0)
            idx_v[g, pl.ds(j * L, L)] = v

    gsems = sems[:NBUF]
    osems = sems[NBUF:]
    gathers = [None] * NBUF
    writes = [None] * NBUF
    for g in range(N_CHUNKS):
        b = g % NBUF
        if writes[b] is not None:
            writes[b].wait()
            writes[b] = None
        gathers[b] = pltpu.async_copy(
            table_hbm.at[idx_v.at[g]], rows_v.at[b], gsems[b])
        if g >= NBUF - 1:
            pb = (g - (NBUF - 1)) % NBUF
            gathers[pb].wait()
            writes[pb] = pltpu.async_copy(
                rows_v.at[pb],
                out_hbm.at[pl.ds(base + (g - (NBUF - 1)) * CHUNK, CHUNK)],
                osems[pb])
    for t in range(N_CHUNKS - (NBUF - 1), N_CHUNKS):
        tb = t % NBUF
        if writes[tb] is not None:
            writes[tb].wait()
            writes[tb] = None
        gathers[tb].wait()
        writes[tb] = pltpu.async_copy(
            rows_v.at[tb], out_hbm.at[pl.ds(base + t * CHUNK, CHUNK)],
            osems[tb])
    for b in range(NBUF):
        if writes[b] is not None:
            writes[b].wait()


@jax.jit
def kernel(position_ids, table):
    idx = position_ids.reshape(NW, N_CHUNKS, CHUNK).astype(jnp.int32)
    run = pl.kernel(
        _gather_kernel,
        out_type=jax.ShapeDtypeStruct((B_TOTAL, D), jnp.float32),
        mesh=plsc.VectorSubcoreMesh(core_axis_name="c", subcore_axis_name="s"),
        scratch_types=[
            pltpu.VMEM((N_CHUNKS, CHUNK), jnp.int32),
            pltpu.VMEM((NBUF, CHUNK, D), jnp.float32),
        ] + [pltpu.SemaphoreType.DMA] * (2 * NBUF),
    )
    out = run(table, idx)
    return out.reshape(position_ids.shape[0], position_ids.shape[1], D)
